# baseline (speedup 1.0000x reference)
import functools

import jax
import jax.numpy as jnp
from jax import lax
from jax.experimental import pallas as pl
from jax.experimental.pallas import tpu as pltpu
from jax.experimental.pallas import tpu_sc as plsc

_B = 16384
_D = 64
_NE = 1000000
_NC = 2
_NS = 16
_NW = _NC * _NS
_NI = 2 * _B
_IPW = _NI // _NW
_BPW = _B // _NW
_RING = 8
_TAIL0 = (_NE // 128) * 128
_LASTB = _TAIL0 // 128 - 1


def _phase1_body(pk_hbm, pd_hbm, fstart_hbm, prime_hbm, et_hbm,
                 ttab_hbm, rows_hbm,
                 pk_v, pd_v, fstart_v, prime_v, ttab_v, blocks, rowstage,
                 bsem, osem):
    wid = lax.axis_index("s") * _NC + lax.axis_index("c")

    pltpu.sync_copy(pk_hbm.at[wid], pk_v)
    pltpu.sync_copy(pd_hbm.at[wid], pd_v)
    pltpu.sync_copy(fstart_hbm.at[wid], fstart_v)
    pltpu.sync_copy(prime_hbm.at[wid], prime_v)
    pltpu.sync_copy(ttab_hbm, ttab_v)

    iota16 = lax.iota(jnp.int32, 16)

    pv = prime_v[pl.ds(0, 16)]
    for s in range(_RING - 1):
        st = pl.multiple_of(pv[s], 128)
        pltpu.async_copy(et_hbm.at[:, pl.ds(st, 128)], blocks.at[s], bsem)

    dummy_blk = et_hbm.at[:, pl.ds(0, 128)]

    def group(g, carry):
        sl = pl.ds(g * 16, 16)
        pk16, pd16, fst16 = pk_v[sl], pd_v[sl], fstart_v[sl]
        for i in range(16):
            pk, pd, fs = pk16[i], pd16[i], fst16[i]
            ln, sm = pk & 127, (pk >> 7) & 7
            fl, nf = (pk >> 10) & 7, (pk >> 13) & 1
            tl = (pk >> 14) & 1
            dn, tb = pd & 32767, pd >> 15

            @pl.when(nf != 0)
            def _():
                pltpu.make_async_copy(dummy_blk, blocks.at[0], bsem).wait()
                st = pl.multiple_of(fs, 128)
                pltpu.async_copy(et_hbm.at[:, pl.ds(st, 128)],
                                 blocks.at[fl], bsem)

            smv = jnp.full((16,), sm, jnp.int32)
            lnv = jnp.full((16,), ln, jnp.int32)
            for c in range(_D // 16):
                dvec = iota16 + c * 16
                bv = plsc.load_gather(blocks, [smv, dvec, lnv])
                tv = plsc.load_gather(ttab_v, [tb + c * 16 + iota16])
                rowstage[i, pl.ds(c * 16, 16)] = jnp.where(tl != 0, tv, bv)
            pltpu.async_copy(rowstage.at[i], rows_hbm.at[dn], osem)
        pltpu.make_async_copy(rows_hbm.at[pl.ds(0, 16)],
                              rowstage, osem).wait()
        return carry

    lax.fori_loop(0, _IPW // 16, group, 0)

    for _ in range(_RING - 1):
        pltpu.make_async_copy(dummy_blk, blocks.at[0], bsem).wait()


_phase1 = functools.partial(
    pl.kernel,
    out_type=jax.ShapeDtypeStruct((_NI, _D), jnp.float32),
    scratch_types=[
        pltpu.VMEM((_IPW,), jnp.int32),
        pltpu.VMEM((_IPW,), jnp.int32),
        pltpu.VMEM((_IPW,), jnp.int32),
        pltpu.VMEM((16,), jnp.int32),
        pltpu.VMEM((4096,), jnp.float32),
        pltpu.VMEM((_RING, _D, 128), jnp.float32),
        pltpu.VMEM((16, _D), jnp.float32),
        pltpu.SemaphoreType.DMA,
        pltpu.SemaphoreType.DMA,
    ],
    mesh=plsc.VectorSubcoreMesh(core_axis_name="c", subcore_axis_name="s"),
    compiler_params=pltpu.CompilerParams(needs_layout_passes=False),
)(_phase1_body)


def _phase2_body(rpair_hbm, rhalf_hbm, rows_hbm, relp_hbm, out_hbm,
                 rpair, rhalf, hbuf, tbuf, rbuf, out_buf, sem):
    wid = lax.axis_index("s") * _NC + lax.axis_index("c")

    pltpu.sync_copy(rpair_hbm.at[pl.ds(wid * 4, 4)], rpair)
    pltpu.sync_copy(rhalf_hbm.at[wid], rhalf)

    iota16 = lax.iota(jnp.int32, 16)

    def do_pass(p, carry):
        row0 = p * 128
        cp_h = pltpu.async_copy(
            rows_hbm.at[pl.ds(wid * _BPW + row0, 128)], hbuf, sem)
        cp_t = pltpu.async_copy(
            rows_hbm.at[pl.ds(_B + wid * _BPW + row0, 128)], tbuf, sem)
        cp_r = pltpu.async_copy(relp_hbm.at[rpair.at[p]], rbuf, sem)
        cp_h.wait()
        cp_t.wait()
        cp_r.wait()
        for g in range(8):
            sl16 = pl.ds(row0 + g * 16, 16)
            rh = rhalf[sl16]
            tot = jnp.zeros((16,), jnp.float32)
            for jj in range(16):
                lr = g * 16 + jj
                ro = rh[jj] * _D
                acc = None
                for c in range(_D // 16):
                    o = c * 16
                    prod = (hbuf[lr, pl.ds(o, 16)]
                            * rbuf[lr, pl.ds(ro + o, 16)]
                            * tbuf[lr, pl.ds(o, 16)])
                    acc = prod if acc is None else acc + prod
                tot = jnp.where(iota16 == jj, jnp.sum(acc), tot)
            out_buf[sl16] = tot
        return carry

    lax.fori_loop(0, _BPW // 128, do_pass, 0)

    pltpu.sync_copy(out_buf, out_hbm.at[pl.ds(wid * _BPW, _BPW)])


_phase2 = functools.partial(
    pl.kernel,
    out_type=jax.ShapeDtypeStruct((_B,), jnp.float32),
    scratch_types=[
        pltpu.VMEM((4, 128), jnp.int32),
        pltpu.VMEM((_BPW,), jnp.int32),
        pltpu.VMEM((128, _D), jnp.float32),
        pltpu.VMEM((128, _D), jnp.float32),
        pltpu.VMEM((128, 2 * _D), jnp.float32),
        pltpu.VMEM((_BPW,), jnp.float32),
        pltpu.SemaphoreType.DMA,
    ],
    mesh=plsc.VectorSubcoreMesh(core_axis_name="c", subcore_axis_name="s"),
    compiler_params=pltpu.CompilerParams(needs_layout_passes=False),
)(_phase2_body)


@jax.jit
def kernel(head_idx, rel_idx, tail_idx, entity_table, relation_table):
    i32 = jnp.int32
    e_all = jnp.concatenate([head_idx, tail_idx]).astype(i32)
    pos = jnp.arange(_NI, dtype=i32)
    se, order = lax.sort((e_all, pos), num_keys=1)
    blk = se >> 7
    first = (pos & (_IPW - 1)) == 0
    prev_blk = jnp.concatenate([blk[:1] - 1, blk[:-1]])
    newf = (first | (blk != prev_blk)).astype(i32)
    bstart = (jnp.minimum(blk, _LASTB) << 7).astype(i32)
    slot = jnp.cumsum(newf, dtype=i32) - 1
    slot0 = jnp.broadcast_to(slot.reshape(_NW, _IPW)[:, :1],
                             (_NW, _IPW)).reshape(-1)
    lslot = slot - slot0
    prime_q = (slot0.reshape(_NW, _IPW)[:, 0:1]
               + jnp.arange(16, dtype=i32)[None, :]).reshape(-1)
    q_all = jnp.searchsorted(slot, jnp.concatenate(
        [slot + (_RING - 1), prime_q])).astype(i32)
    q_all = jnp.minimum(q_all, _NI - 1)
    starts = bstart[q_all]
    fstart = starts[:_NI]
    prime = starts[_NI:].reshape(_NW, 16)
    fslot = (lslot + _RING - 1) & (_RING - 1)
    istail = se >= _TAIL0
    tk = se - _TAIL0
    tbase = jnp.where(istail, ((tk >> 1) << 7) + ((tk & 1) << 6), 0)

    slotm = lslot & (_RING - 1)
    pk = ((se & 127) | (slotm << 7) | (fslot << 10) | (newf << 13)
          | (istail.astype(i32) << 14))
    pd = order | (tbase << 15)
    shape_w = (_NW, _IPW)
    ttab = entity_table[_TAIL0:].reshape(32, 128)
    rows = _phase1(pk.reshape(shape_w), pd.reshape(shape_w),
                   fstart.reshape(shape_w), prime,
                   entity_table.T, ttab.reshape(-1))

    r2 = rel_idx.astype(i32).reshape(_NW, _BPW)
    relp = relation_table.reshape(500, 2 * _D)
    return _phase2((r2 >> 1).reshape(_NW * 4, 128), r2 & 1, rows, relp)


# speedup vs baseline: 2.2966x; 2.2966x over previous
import functools

import jax
import jax.numpy as jnp
from jax import lax
from jax.experimental import pallas as pl
from jax.experimental.pallas import tpu as pltpu
from jax.experimental.pallas import tpu_sc as plsc

_B = 16384
_D = 64
_NE = 1000000
_NC = 2
_NS = 16
_NW = _NC * _NS
_NI = 2 * _B
_IPW = _NI // _NW
_BPW = _B // _NW
_RING = 8
_TAIL0 = (_NE // 128) * 128
_LASTB = _TAIL0 // 128 - 1


def _phase1_body(pk_hbm, pd_hbm, fstart_hbm, prime_hbm, et_hbm,
                 ttab_hbm, rows_hbm,
                 pk_v, pd_v, fstart_v, prime_v, ttab_v, blocks, rowstage,
                 bsem, osem):
    wid = lax.axis_index("s") * _NC + lax.axis_index("c")

    pltpu.sync_copy(pk_hbm.at[wid], pk_v)
    pltpu.sync_copy(pd_hbm.at[wid], pd_v)
    pltpu.sync_copy(fstart_hbm.at[wid], fstart_v)
    pltpu.sync_copy(prime_hbm.at[wid], prime_v)
    pltpu.sync_copy(ttab_hbm, ttab_v)

    iota16 = lax.iota(jnp.int32, 16)

    pv = prime_v[pl.ds(0, 16)]
    for s in range(_RING - 1):
        st = pl.multiple_of(pv[s], 128)
        pltpu.async_copy(et_hbm.at[:, pl.ds(st, 128)], blocks.at[s], bsem)

    dummy_blk = et_hbm.at[:, pl.ds(0, 128)]

    def group(g, carry):
        sl = pl.ds(g * 16, 16)
        pk16, pd16, fst16 = pk_v[sl], pd_v[sl], fstart_v[sl]
        for i in range(16):
            pk, pd, fs = pk16[i], pd16[i], fst16[i]
            ln, sm = pk & 127, (pk >> 7) & 7
            fl, nf = (pk >> 10) & 7, (pk >> 13) & 1
            tl = (pk >> 14) & 1
            dn, tb = pd & 32767, pd >> 15

            @pl.when(nf != 0)
            def _():
                pltpu.make_async_copy(dummy_blk, blocks.at[0], bsem).wait()
                st = pl.multiple_of(fs, 128)
                pltpu.async_copy(et_hbm.at[:, pl.ds(st, 128)],
                                 blocks.at[fl], bsem)

            smv = jnp.full((16,), sm, jnp.int32)
            lnv = jnp.full((16,), ln, jnp.int32)
            for c in range(_D // 16):
                dvec = iota16 + c * 16
                bv = plsc.load_gather(blocks, [smv, dvec, lnv])
                tv = plsc.load_gather(ttab_v, [tb + c * 16 + iota16])
                rowstage[i, pl.ds(c * 16, 16)] = jnp.where(tl != 0, tv, bv)
            pltpu.async_copy(rowstage.at[i], rows_hbm.at[dn], osem)
        pltpu.make_async_copy(rows_hbm.at[pl.ds(0, 16)],
                              rowstage, osem).wait()
        return carry

    lax.fori_loop(0, _IPW // 16, group, 0)

    for _ in range(_RING - 1):
        pltpu.make_async_copy(dummy_blk, blocks.at[0], bsem).wait()


_phase1 = functools.partial(
    pl.kernel,
    out_type=jax.ShapeDtypeStruct((_NI, _D), jnp.float32),
    scratch_types=[
        pltpu.VMEM((_IPW,), jnp.int32),
        pltpu.VMEM((_IPW,), jnp.int32),
        pltpu.VMEM((_IPW,), jnp.int32),
        pltpu.VMEM((16,), jnp.int32),
        pltpu.VMEM((4096,), jnp.float32),
        pltpu.VMEM((_RING, _D, 128), jnp.float32),
        pltpu.VMEM((16, _D), jnp.float32),
        pltpu.SemaphoreType.DMA,
        pltpu.SemaphoreType.DMA,
    ],
    mesh=plsc.VectorSubcoreMesh(core_axis_name="c", subcore_axis_name="s"),
    compiler_params=pltpu.CompilerParams(needs_layout_passes=False),
)(_phase1_body)


def _phase2_body(rpair_hbm, rhalf_hbm, rows_hbm, relp_hbm, out_hbm,
                 rpair, rhalf, hbuf, tbuf, rbuf, out_buf, sem):
    wid = lax.axis_index("s") * _NC + lax.axis_index("c")

    pltpu.sync_copy(rpair_hbm.at[pl.ds(wid * 4, 4)], rpair)
    pltpu.sync_copy(rhalf_hbm.at[wid], rhalf)

    iota16 = lax.iota(jnp.int32, 16)

    def do_pass(p, carry):
        row0 = p * 128
        cp_h = pltpu.async_copy(
            rows_hbm.at[pl.ds(wid * _BPW + row0, 128)], hbuf, sem)
        cp_t = pltpu.async_copy(
            rows_hbm.at[pl.ds(_B + wid * _BPW + row0, 128)], tbuf, sem)
        cp_r = pltpu.async_copy(relp_hbm.at[rpair.at[p]], rbuf, sem)
        cp_h.wait()
        cp_t.wait()
        cp_r.wait()
        for g in range(8):
            sl16 = pl.ds(row0 + g * 16, 16)
            rh = rhalf[sl16]
            tot = jnp.zeros((16,), jnp.float32)
            for jj in range(16):
                lr = g * 16 + jj
                ro = rh[jj] * _D
                acc = None
                for c in range(_D // 16):
                    o = c * 16
                    prod = (hbuf[lr, pl.ds(o, 16)]
                            * rbuf[lr, pl.ds(ro + o, 16)]
                            * tbuf[lr, pl.ds(o, 16)])
                    acc = prod if acc is None else acc + prod
                tot = jnp.where(iota16 == jj, jnp.sum(acc), tot)
            out_buf[sl16] = tot
        return carry

    lax.fori_loop(0, _BPW // 128, do_pass, 0)

    pltpu.sync_copy(out_buf, out_hbm.at[pl.ds(wid * _BPW, _BPW)])


_phase2 = functools.partial(
    pl.kernel,
    out_type=jax.ShapeDtypeStruct((_B,), jnp.float32),
    scratch_types=[
        pltpu.VMEM((4, 128), jnp.int32),
        pltpu.VMEM((_BPW,), jnp.int32),
        pltpu.VMEM((128, _D), jnp.float32),
        pltpu.VMEM((128, _D), jnp.float32),
        pltpu.VMEM((128, 2 * _D), jnp.float32),
        pltpu.VMEM((_BPW,), jnp.float32),
        pltpu.SemaphoreType.DMA,
    ],
    mesh=plsc.VectorSubcoreMesh(core_axis_name="c", subcore_axis_name="s"),
    compiler_params=pltpu.CompilerParams(needs_layout_passes=False),
)(_phase2_body)


@jax.jit
def kernel(head_idx, rel_idx, tail_idx, entity_table, relation_table):
    i32 = jnp.int32
    e_all = jnp.concatenate([head_idx, tail_idx]).astype(i32)
    pos = jnp.arange(_NI, dtype=i32)
    se, order = lax.sort((e_all, pos), num_keys=1)
    blk = se >> 7
    first = (pos & (_IPW - 1)) == 0
    prev_blk = jnp.concatenate([blk[:1] - 1, blk[:-1]])
    newf = (first | (blk != prev_blk)).astype(i32)
    bstart = (jnp.minimum(blk, _LASTB) << 7).astype(i32)
    slot = jnp.cumsum(newf, dtype=i32) - 1
    slot0 = jnp.broadcast_to(slot.reshape(_NW, _IPW)[:, :1],
                             (_NW, _IPW)).reshape(-1)
    lslot = slot - slot0
    wk = pos >> 10
    blist = jnp.zeros((_NW, _IPW), i32).at[wk, lslot].set(bstart)
    fstart = blist[wk, jnp.minimum(lslot + _RING - 1, _IPW - 1)]
    prime = blist[:, :16]
    fslot = (lslot + _RING - 1) & (_RING - 1)
    istail = se >= _TAIL0
    tk = se - _TAIL0
    tbase = jnp.where(istail, ((tk >> 1) << 7) + ((tk & 1) << 6), 0)

    slotm = lslot & (_RING - 1)
    pk = ((se & 127) | (slotm << 7) | (fslot << 10) | (newf << 13)
          | (istail.astype(i32) << 14))
    pd = order | (tbase << 15)
    shape_w = (_NW, _IPW)
    ttab = entity_table[_TAIL0:].reshape(32, 128)
    rows = _phase1(pk.reshape(shape_w), pd.reshape(shape_w),
                   fstart.reshape(shape_w), prime,
                   entity_table.T, ttab.reshape(-1))

    r2 = rel_idx.astype(i32).reshape(_NW, _BPW)
    relp = relation_table.reshape(500, 2 * _D)
    return _phase2((r2 >> 1).reshape(_NW * 4, 128), r2 & 1, rows, relp)


# speedup vs baseline: 2.6017x; 1.1328x over previous
import functools

import jax
import jax.numpy as jnp
from jax import lax
from jax.experimental import pallas as pl
from jax.experimental.pallas import tpu as pltpu
from jax.experimental.pallas import tpu_sc as plsc

_B = 16384
_D = 64
_TR = 8
_NTILES = 1000000 // _TR
_NC = 2
_NS = 16
_NW = _NC * _NS
_BPW = _B // _NW
_PASS = 128
_NPASS = _BPW // _PASS


def _distmult_body(hidx_hbm, tidx_hbm, ridx_hbm, entity_hbm, rel_hbm,
                   out_hbm, hidx, tidx, ridx,
                   h_rows, t_rows, r_rows, out_buf, sem):
    wid = lax.axis_index("s") * _NC + lax.axis_index("c")

    pltpu.sync_copy(hidx_hbm.at[wid], hidx)
    pltpu.sync_copy(tidx_hbm.at[wid], tidx)
    pltpu.sync_copy(ridx_hbm.at[wid], ridx)

    iota16 = lax.iota(jnp.int32, 16)

    def do_pass(p, carry):
        row0 = p * _PASS
        for half in range(_PASS // 64):
            for q in range(4):
                base = row0 + half * 64 + q * 16
                sl = pl.ds(base, 16)
                hv, tv, rv = hidx[sl], tidx[sl], ridx[sl]
                for i in range(16):
                    lj = half * 64 + q * 16 + i
                    a, b = lj >> 3, lj & 7
                    hh, tt, rr = hv[i], tv[i], rv[i]
                    pltpu.async_copy(entity_hbm.at[hh >> 3, hh & 7],
                                     h_rows.at[a, b], sem)
                    pltpu.async_copy(entity_hbm.at[tt >> 3, tt & 7],
                                     t_rows.at[a, b], sem)
                    pltpu.async_copy(rel_hbm.at[rr >> 3, rr & 7],
                                     r_rows.at[a, b], sem)
            blk = pl.ds(half * 8, 8)
            src = entity_hbm.at[pl.ds(0, 8)]
            rsrc = rel_hbm.at[pl.ds(0, 8)]
            pltpu.make_async_copy(src, h_rows.at[blk], sem).wait()
            pltpu.make_async_copy(src, t_rows.at[blk], sem).wait()
            pltpu.make_async_copy(rsrc, r_rows.at[blk], sem).wait()

        for g in range(_PASS // 16):
            tot = jnp.zeros((16,), jnp.float32)
            for jj in range(16):
                lj = g * 16 + jj
                a, b = lj >> 3, lj & 7
                acc = None
                for c in range(_D // 16):
                    sl = pl.ds(c * 16, 16)
                    prod = (h_rows[a, b, sl] * r_rows[a, b, sl]
                            * t_rows[a, b, sl])
                    acc = prod if acc is None else acc + prod
                tot = jnp.where(iota16 == jj, jnp.sum(acc), tot)
            out_buf[pl.ds(row0 + g * 16, 16)] = tot
        return carry

    lax.fori_loop(0, _NPASS, do_pass, 0)

    pltpu.sync_copy(out_buf, out_hbm.at[pl.ds(wid * _BPW, _BPW)])


_distmult_sc = functools.partial(
    pl.kernel,
    out_type=jax.ShapeDtypeStruct((_B,), jnp.float32),
    scratch_types=[
        pltpu.VMEM((_BPW,), jnp.int32),
        pltpu.VMEM((_BPW,), jnp.int32),
        pltpu.VMEM((_BPW,), jnp.int32),
        pltpu.VMEM((_PASS // _TR, _TR, _D), jnp.float32),
        pltpu.VMEM((_PASS // _TR, _TR, _D), jnp.float32),
        pltpu.VMEM((_PASS // _TR, _TR, _D), jnp.float32),
        pltpu.VMEM((_BPW,), jnp.float32),
        pltpu.SemaphoreType.DMA,
    ],
    mesh=plsc.VectorSubcoreMesh(core_axis_name="c", subcore_axis_name="s"),
    compiler_params=pltpu.CompilerParams(needs_layout_passes=False),
)(_distmult_body)


@jax.jit
def kernel(head_idx, rel_idx, tail_idx, entity_table, relation_table):
    h2, r2, t2 = (x.reshape(_NW, _BPW) for x in (head_idx, rel_idx, tail_idx))
    et3 = entity_table.reshape(_NTILES, _TR, _D)
    rt3 = relation_table.reshape(1000 // _TR, _TR, _D)
    return _distmult_sc(h2, t2, r2, et3, rt3)


# speedup vs baseline: 2.6685x; 1.0257x over previous
import functools

import jax
import jax.numpy as jnp
from jax import lax
from jax.experimental import pallas as pl
from jax.experimental.pallas import tpu as pltpu
from jax.experimental.pallas import tpu_sc as plsc

_B = 16384
_D = 64
_TR = 8
_NTILES = 1000000 // _TR
_NC = 2
_NS = 16
_NW = _NC * _NS
_BPW = _B // _NW
_PASS = 128
_NPASS = _BPW // _PASS


def _distmult_body(hidx_hbm, tidx_hbm, ridx_hbm, entity_hbm, rel_hbm,
                   out_hbm, hidx, tidx, ridx,
                   h_rows, t_rows, r_rows, out_buf, sem):
    wid = lax.axis_index("s") * _NC + lax.axis_index("c")

    pltpu.sync_copy(hidx_hbm.at[wid], hidx)
    pltpu.sync_copy(tidx_hbm.at[wid], tidx)
    pltpu.sync_copy(ridx_hbm.at[wid], ridx)

    iota16 = lax.iota(jnp.int32, 16)

    def issue(p, par):
        row0 = p * _PASS
        for q in range(_PASS // 16):
            sl = pl.ds(row0 + q * 16, 16)
            hv, tv, rv = hidx[sl], tidx[sl], ridx[sl]
            for i in range(16):
                lj = q * 16 + i
                a, b = lj >> 3, lj & 7
                hh, tt, rr = hv[i], tv[i], rv[i]
                pltpu.async_copy(entity_hbm.at[hh >> 3, hh & 7],
                                 h_rows.at[par, a, b], sem)
                pltpu.async_copy(entity_hbm.at[tt >> 3, tt & 7],
                                 t_rows.at[par, a, b], sem)
                pltpu.async_copy(rel_hbm.at[rr >> 3, rr & 7],
                                 r_rows.at[par, a, b], sem)

    issue(0, 0)

    def do_pass(p, carry):
        par = p & 1

        @pl.when(p < _NPASS - 1)
        def _():
            issue(p + 1, 1 - par)

        src = entity_hbm.at[pl.ds(0, 16)]
        rsrc = rel_hbm.at[pl.ds(0, 16)]
        pltpu.make_async_copy(src, h_rows.at[par], sem).wait()
        pltpu.make_async_copy(src, t_rows.at[par], sem).wait()
        pltpu.make_async_copy(rsrc, r_rows.at[par], sem).wait()

        row0 = p * _PASS
        for g in range(_PASS // 16):
            tot = jnp.zeros((16,), jnp.float32)
            for jj in range(16):
                lj = g * 16 + jj
                a, b = lj >> 3, lj & 7
                acc = None
                for c in range(_D // 16):
                    sl = pl.ds(c * 16, 16)
                    prod = (h_rows[par, a, b, sl] * r_rows[par, a, b, sl]
                            * t_rows[par, a, b, sl])
                    acc = prod if acc is None else acc + prod
                tot = jnp.where(iota16 == jj, jnp.sum(acc), tot)
            out_buf[pl.ds(row0 + g * 16, 16)] = tot
        return carry

    lax.fori_loop(0, _NPASS, do_pass, 0)

    pltpu.sync_copy(out_buf, out_hbm.at[pl.ds(wid * _BPW, _BPW)])


_distmult_sc = functools.partial(
    pl.kernel,
    out_type=jax.ShapeDtypeStruct((_B,), jnp.float32),
    scratch_types=[
        pltpu.VMEM((_BPW,), jnp.int32),
        pltpu.VMEM((_BPW,), jnp.int32),
        pltpu.VMEM((_BPW,), jnp.int32),
        pltpu.VMEM((2, _PASS // _TR, _TR, _D), jnp.float32),
        pltpu.VMEM((2, _PASS // _TR, _TR, _D), jnp.float32),
        pltpu.VMEM((2, _PASS // _TR, _TR, _D), jnp.float32),
        pltpu.VMEM((_BPW,), jnp.float32),
        pltpu.SemaphoreType.DMA,
    ],
    mesh=plsc.VectorSubcoreMesh(core_axis_name="c", subcore_axis_name="s"),
    compiler_params=pltpu.CompilerParams(needs_layout_passes=False),
)(_distmult_body)


@jax.jit
def kernel(head_idx, rel_idx, tail_idx, entity_table, relation_table):
    h2, r2, t2 = (x.reshape(_NW, _BPW) for x in (head_idx, rel_idx, tail_idx))
    et3 = entity_table.reshape(_NTILES, _TR, _D)
    rt3 = relation_table.reshape(1000 // _TR, _TR, _D)
    return _distmult_sc(h2, t2, r2, et3, rt3)
